# Initial kernel scaffold; baseline (speedup 1.0000x reference)
import jax
import jax.numpy as jnp
from jax.experimental import pallas as pl


def kernel(question_words, question_chars, question_len, word_table, char_table):
    raise NotImplementedError("write your pallas kernel here")


# speedup vs baseline: 9.0796x; 9.0796x over previous
import jax
import jax.numpy as jnp
from jax import lax
from jax.experimental import pallas as pl
from jax.experimental.pallas import tpu as pltpu
from jax.experimental.pallas import tpu_sc as plsc

_B, _Q, _W = 4096, 50, 16
_DW, _DC = 64, 16
_VC = 512
_BQ = _B * _Q
_NW = 32
_PER_W = _BQ // _NW
_CHUNK = 128
_NCHUNK = _PER_W // _CHUNK
_L = 16


def _splat(v, dtype=jnp.int32):
    return jnp.full((_L,), v, dtype)


def _sc_body(qw_hbm, qc_hbm, qlen_hbm, wtab_hbm, ctab_hbm, out_hbm,
             ctab_v, len_v, widx_v, wbuf_v, cidx_v, cbuf_v, sem):
    nc = 2
    wid = lax.axis_index("s") * nc + lax.axis_index("c")
    my_base = wid * _PER_W

    pltpu.sync_copy(ctab_hbm, ctab_v)
    pltpu.sync_copy(qlen_hbm, len_v)

    lane = lax.iota(jnp.int32, _L)
    zeros = jnp.zeros((_L,), jnp.float32)

    def chunk_body(c, carry):
        base = my_base + c * _CHUNK
        pltpu.sync_copy(qw_hbm.at[pl.ds(base, _CHUNK)], widx_v)
        gather = pltpu.async_copy(wtab_hbm.at[widx_v], wbuf_v, sem)
        pltpu.sync_copy(qc_hbm.at[pl.ds(base, _CHUNK), :], cidx_v)
        gather.wait()

        def group_body(g, carry2):
            pos = g * _L + lane
            gpos = base + pos
            gf = gpos.astype(jnp.float32)
            b = ((gf + 0.5) * (1.0 / _Q)).astype(jnp.int32)
            q = gpos - b * _Q
            ln = plsc.load_gather(len_v, [b])
            valid = q < ln
            maskf = jnp.where(valid, 1.0, 0.0).astype(jnp.float32)

            acc = [None] * _DC
            for w in range(_W):
                cw = plsc.load_gather(cidx_v, [pos, _splat(w)])
                for d in range(_DC):
                    v = plsc.load_gather(ctab_v, [cw, _splat(d)])
                    acc[d] = v if acc[d] is None else jnp.maximum(acc[d], v)
            for d in range(_DC):
                plsc.store_scatter(cbuf_v, [pos, _splat(d)], acc[d] * maskf)

            invalid = jnp.logical_not(valid)
            for k in range(_DW):
                plsc.store_scatter(wbuf_v, [pos, _splat(k)], zeros,
                                   mask=invalid)
            return carry2

        lax.fori_loop(0, _CHUNK // _L, group_body, 0)

        pltpu.sync_copy(cbuf_v, out_hbm.at[pl.ds(base, _CHUNK), pl.ds(0, _DC)])
        pltpu.sync_copy(wbuf_v, out_hbm.at[pl.ds(base, _CHUNK), pl.ds(_DC, _DW)])
        return carry

    lax.fori_loop(0, _NCHUNK, chunk_body, 0)


_sc_call = pl.kernel(
    _sc_body,
    out_type=jax.ShapeDtypeStruct((_BQ, _DC + _DW), jnp.float32),
    mesh=plsc.VectorSubcoreMesh(core_axis_name="c", subcore_axis_name="s"),
    scratch_types=[
        pltpu.VMEM((_VC, _DC), jnp.float32),
        pltpu.VMEM((_B,), jnp.int32),
        pltpu.VMEM((_CHUNK,), jnp.int32),
        pltpu.VMEM((_CHUNK, _DW), jnp.float32),
        pltpu.VMEM((_CHUNK, _W), jnp.int32),
        pltpu.VMEM((_CHUNK, _DC), jnp.float32),
        pltpu.SemaphoreType.DMA,
    ],
    compiler_params=pltpu.CompilerParams(use_tc_tiling_on_sc=False,
                                         needs_layout_passes=False),
)


def kernel(question_words, question_chars, question_len, word_table, char_table):
    qw = question_words.reshape(_BQ).astype(jnp.int32)
    qc = question_chars.reshape(_BQ, _W).astype(jnp.int32)
    ql = question_len.astype(jnp.int32)
    out = _sc_call(qw, qc, ql, word_table, char_table)
    return out.reshape(_B, _Q, _DC + _DW)


# speedup vs baseline: 10.1348x; 1.1162x over previous
import jax
import jax.numpy as jnp
from jax import lax
from jax.experimental import pallas as pl
from jax.experimental.pallas import tpu as pltpu
from jax.experimental.pallas import tpu_sc as plsc

_B, _Q, _W = 4096, 50, 16
_DW, _DC = 64, 16
_DOUT = _DC + _DW
_VC = 512
_BQ = _B * _Q
_NW = 32
_PER_W = _BQ // _NW
_CHUNK = 128
_NCHUNK = _PER_W // _CHUNK
_R = 5
_NITER = _NCHUNK // _R
_L = 16


def _splat(v, dtype=jnp.int32):
    return jnp.full((_L,), v, dtype)


def _sc_body(qw_hbm, qc_hbm, qlen_hbm, wtab_hbm, ctab_hbm, out_hbm,
             ctab_v, len_v, widx_v, cidx_v, wbuf_v, cbuf_v,
             sem_g, sem_ci, sem_ow, sem_oc):
    nc = 2
    wid = lax.axis_index("s") * nc + lax.axis_index("c")
    my_base = wid * _PER_W

    pltpu.sync_copy(ctab_hbm, ctab_v)
    pltpu.sync_copy(qlen_hbm, len_v)
    pltpu.sync_copy(qw_hbm.at[pl.ds(my_base, _PER_W)], widx_v)

    lane = lax.iota(jnp.int32, _L)
    zeros = jnp.zeros((_L,), jnp.float32)

    def fire_inputs(c, j):
        base = my_base + c * _CHUNK
        pltpu.async_copy(qc_hbm.at[pl.ds(base, _CHUNK), :], cidx_v.at[j],
                         sem_ci.at[j])
        pltpu.async_copy(
            wtab_hbm.at[widx_v.at[pl.ds(c * _CHUNK, _CHUNK)]],
            wbuf_v.at[j], sem_g.at[j])

    def drain_inputs(j):
        pltpu.make_async_copy(qc_hbm.at[pl.ds(0, _CHUNK), :], cidx_v.at[j],
                              sem_ci.at[j]).wait()
        pltpu.make_async_copy(wtab_hbm.at[pl.ds(0, _CHUNK), :],
                              wbuf_v.at[j], sem_g.at[j]).wait()

    def fire_out(c, j):
        base = my_base + c * _CHUNK
        pltpu.async_copy(cbuf_v.at[j],
                         out_hbm.at[pl.ds(base, _CHUNK), pl.ds(0, _DC)],
                         sem_oc.at[j])
        pltpu.async_copy(wbuf_v.at[j],
                         out_hbm.at[pl.ds(base, _CHUNK), pl.ds(_DC, _DW)],
                         sem_ow.at[j])

    def drain_out(j):
        pltpu.make_async_copy(cbuf_v.at[j],
                              out_hbm.at[pl.ds(0, _CHUNK), pl.ds(0, _DC)],
                              sem_oc.at[j]).wait()
        pltpu.make_async_copy(wbuf_v.at[j],
                              out_hbm.at[pl.ds(0, _CHUNK), pl.ds(_DC, _DW)],
                              sem_ow.at[j]).wait()

    def compute_chunk(base, cidx, wbuf, cbuf):
        def group_body(g, carry):
            pos = g * _L + lane
            gpos = base + pos
            gf = gpos.astype(jnp.float32)
            b = ((gf + 0.5) * (1.0 / _Q)).astype(jnp.int32)
            q = gpos - b * _Q
            ln = plsc.load_gather(len_v, [b])
            valid = q < ln
            maskf = jnp.where(valid, 1.0, 0.0).astype(jnp.float32)

            acc = [None] * _DC
            for w in range(_W):
                cw = plsc.load_gather(cidx, [pos, _splat(w)])
                for d in range(_DC):
                    v = plsc.load_gather(ctab_v, [cw, _splat(d)])
                    acc[d] = v if acc[d] is None else jnp.maximum(acc[d], v)
            for d in range(_DC):
                plsc.store_scatter(cbuf, [pos, _splat(d)], acc[d] * maskf)

            invalid = jnp.logical_not(valid)
            for k in range(_DW):
                plsc.store_scatter(wbuf, [pos, _splat(k)], zeros,
                                   mask=invalid)
            return carry

        lax.fori_loop(0, _CHUNK // _L, group_body, 0)

    fire_inputs(0, 0)

    def iter_body(i, carry):
        for j in range(_R):
            c = i * _R + j
            nj = (j + 1) % _R
            if j == _R - 1:
                drain_out(nj)
                @pl.when(i < _NITER - 1)
                def _fire():
                    fire_inputs(c + 1, nj)
            else:
                @pl.when(i > 0)
                def _drain():
                    drain_out(nj)
                fire_inputs(c + 1, nj)
            drain_inputs(j)
            compute_chunk(my_base + c * _CHUNK, cidx_v.at[j],
                          wbuf_v.at[j], cbuf_v.at[j])
            fire_out(c, j)
        return carry

    lax.fori_loop(0, _NITER, iter_body, 0)

    for j in range(1, _R):
        drain_out(j)


_sc_call = pl.kernel(
    _sc_body,
    out_type=jax.ShapeDtypeStruct((_BQ, _DOUT), jnp.float32),
    mesh=plsc.VectorSubcoreMesh(core_axis_name="c", subcore_axis_name="s"),
    scratch_types=[
        pltpu.VMEM((_VC, _DC), jnp.float32),
        pltpu.VMEM((_B,), jnp.int32),
        pltpu.VMEM((_PER_W,), jnp.int32),
        pltpu.VMEM((_R, _CHUNK, _W), jnp.int32),
        pltpu.VMEM((_R, _CHUNK, _DW), jnp.float32),
        pltpu.VMEM((_R, _CHUNK, _DC), jnp.float32),
        pltpu.SemaphoreType.DMA((_R,)),
        pltpu.SemaphoreType.DMA((_R,)),
        pltpu.SemaphoreType.DMA((_R,)),
        pltpu.SemaphoreType.DMA((_R,)),
    ],
    compiler_params=pltpu.CompilerParams(use_tc_tiling_on_sc=False,
                                         needs_layout_passes=False),
)


def kernel(question_words, question_chars, question_len, word_table, char_table):
    qw = question_words.reshape(_BQ).astype(jnp.int32)
    qc = question_chars.reshape(_BQ, _W).astype(jnp.int32)
    ql = question_len.astype(jnp.int32)
    out = _sc_call(qw, qc, ql, word_table, char_table)
    return out.reshape(_B, _Q, _DOUT)


# speedup vs baseline: 12.0167x; 1.1857x over previous
import jax
import jax.numpy as jnp
from jax import lax
from jax.experimental import pallas as pl
from jax.experimental.pallas import tpu as pltpu
from jax.experimental.pallas import tpu_sc as plsc

_B, _Q, _W = 4096, 50, 16
_DW, _DC = 64, 16
_VC = 512
_NW = 32
_BPW = _B // _NW
_R = 5
_NITER = _Q // _R
_L = 16


def _splat(v, dtype=jnp.int32):
    return jnp.full((_L,), v, dtype)


def _sc_body(qw_hbm, qc_hbm, qlen_hbm, wtab_hbm, ctab_hbm, outc_hbm, outw_hbm,
             ctab_v, len_v, widx_v, cidx_v, wbuf_v, cbuf_v,
             sem_g, sem_ci, sem_ow, sem_oc):
    nc = 2
    wid = lax.axis_index("s") * nc + lax.axis_index("c")
    b0 = wid * _BPW

    pltpu.sync_copy(ctab_hbm, ctab_v)
    pltpu.sync_copy(qlen_hbm.at[pl.ds(b0, _BPW)], len_v)
    pltpu.sync_copy(qw_hbm.at[:, pl.ds(b0, _BPW)], widx_v)

    lane = lax.iota(jnp.int32, _L)
    zeros = jnp.zeros((_L,), jnp.float32)

    def fire_inputs(q, j):
        pltpu.async_copy(qc_hbm.at[q, :, pl.ds(b0, _BPW)], cidx_v.at[j],
                         sem_ci.at[j])
        pltpu.async_copy(wtab_hbm.at[widx_v.at[q]], wbuf_v.at[j], sem_g.at[j])

    def drain_inputs(j):
        pltpu.make_async_copy(qc_hbm.at[0, :, pl.ds(b0, _BPW)], cidx_v.at[j],
                              sem_ci.at[j]).wait()
        pltpu.make_async_copy(wtab_hbm.at[pl.ds(0, _BPW), :],
                              wbuf_v.at[j], sem_g.at[j]).wait()

    def fire_out(q, j):
        pltpu.async_copy(cbuf_v.at[j], outc_hbm.at[q, :, pl.ds(b0, _BPW)],
                         sem_oc.at[j])
        pltpu.async_copy(wbuf_v.at[j], outw_hbm.at[q, pl.ds(b0, _BPW), :],
                         sem_ow.at[j])

    def drain_out(j):
        pltpu.make_async_copy(cbuf_v.at[j], outc_hbm.at[0, :, pl.ds(b0, _BPW)],
                              sem_oc.at[j]).wait()
        pltpu.make_async_copy(wbuf_v.at[j], outw_hbm.at[0, pl.ds(b0, _BPW), :],
                              sem_ow.at[j]).wait()

    def compute_chunk(q, cidx, wbuf, cbuf):
        qv = jnp.full((_L,), q, jnp.int32)

        def group_body(g, carry):
            goff = g * _L
            lenv = len_v[pl.ds(goff, _L)]
            valid = qv < lenv
            maskf = jnp.where(valid, 1.0, 0.0).astype(jnp.float32)

            acc = [None] * _DC
            for w in range(_W):
                cw = cidx[w, pl.ds(goff, _L)]
                for d in range(_DC):
                    v = plsc.load_gather(ctab_v, [_splat(d), cw])
                    acc[d] = v if acc[d] is None else jnp.maximum(acc[d], v)
            for d in range(_DC):
                cbuf[d, pl.ds(goff, _L)] = acc[d] * maskf

            invalid = jnp.logical_not(valid)
            b_rel = goff + lane
            for k in range(_DW):
                plsc.store_scatter(wbuf, [b_rel, _splat(k)], zeros,
                                   mask=invalid)
            return carry

        lax.fori_loop(0, _BPW // _L, group_body, 0)

    fire_inputs(0, 0)

    def iter_body(i, carry):
        for j in range(_R):
            q = i * _R + j
            nj = (j + 1) % _R
            if j == _R - 1:
                drain_out(nj)
                @pl.when(i < _NITER - 1)
                def _fire():
                    fire_inputs(q + 1, nj)
            else:
                @pl.when(i > 0)
                def _drain():
                    drain_out(nj)
                fire_inputs(q + 1, nj)
            drain_inputs(j)
            compute_chunk(q, cidx_v.at[j], wbuf_v.at[j], cbuf_v.at[j])
            fire_out(q, j)
        return carry

    lax.fori_loop(0, _NITER, iter_body, 0)

    for j in range(1, _R):
        drain_out(j)


_sc_call = pl.kernel(
    _sc_body,
    out_type=(
        jax.ShapeDtypeStruct((_Q, _DC, _B), jnp.float32),
        jax.ShapeDtypeStruct((_Q, _B, _DW), jnp.float32),
    ),
    mesh=plsc.VectorSubcoreMesh(core_axis_name="c", subcore_axis_name="s"),
    scratch_types=[
        pltpu.VMEM((_DC, _VC), jnp.float32),
        pltpu.VMEM((_BPW,), jnp.int32),
        pltpu.VMEM((_Q, _BPW), jnp.int32),
        pltpu.VMEM((_R, _W, _BPW), jnp.int32),
        pltpu.VMEM((_R, _BPW, _DW), jnp.float32),
        pltpu.VMEM((_R, _DC, _BPW), jnp.float32),
        pltpu.SemaphoreType.DMA((_R,)),
        pltpu.SemaphoreType.DMA((_R,)),
        pltpu.SemaphoreType.DMA((_R,)),
        pltpu.SemaphoreType.DMA((_R,)),
    ],
    compiler_params=pltpu.CompilerParams(use_tc_tiling_on_sc=False,
                                         needs_layout_passes=False),
)


def kernel(question_words, question_chars, question_len, word_table, char_table):
    qw_t = question_words.astype(jnp.int32).T
    qc_t = question_chars.astype(jnp.int32).transpose(1, 2, 0)
    ctab_t = char_table.T
    ql = question_len.astype(jnp.int32)
    out_c, out_w = _sc_call(qw_t, qc_t, ql, word_table, ctab_t)
    return jnp.concatenate(
        [out_c.transpose(2, 0, 1), out_w.transpose(1, 0, 2)], axis=2)


# speedup vs baseline: 13.0160x; 1.0832x over previous
import jax
import jax.numpy as jnp
from jax import lax
from jax.experimental import pallas as pl
from jax.experimental.pallas import tpu as pltpu
from jax.experimental.pallas import tpu_sc as plsc

_B, _Q, _W = 4096, 50, 16
_DW, _DC = 64, 16
_DWP = 128
_VC = 512
_NW = 32
_BPW = _B // _NW
_R = 5
_NITER = _Q // _R
_L = 16


def _splat(v, dtype=jnp.int32):
    return jnp.full((_L,), v, dtype)


def _sc_body(qw_hbm, qc_hbm, qlen_hbm, wtab_hbm, ctab_hbm, outc_hbm, outw_hbm,
             ctab_v, len_v, widx_v, cidx_v, wbuf_v, cbuf_v,
             sem_g, sem_ci, sem_ow, sem_oc):
    nc = 2
    wid = lax.axis_index("s") * nc + lax.axis_index("c")
    b0 = wid * _BPW

    pltpu.sync_copy(ctab_hbm, ctab_v)
    pltpu.sync_copy(qlen_hbm.at[pl.ds(b0, _BPW)], len_v)
    pltpu.sync_copy(qw_hbm.at[:, pl.ds(b0, _BPW)], widx_v)

    lane = lax.iota(jnp.int32, _L)
    zeros = jnp.zeros((_L,), jnp.float32)

    def fire_inputs(q, j):
        pltpu.async_copy(qc_hbm.at[q, :, pl.ds(b0, _BPW)], cidx_v.at[j],
                         sem_ci.at[j])
        pltpu.async_copy(wtab_hbm.at[widx_v.at[q]], wbuf_v.at[j], sem_g.at[j])

    def drain_inputs(j):
        pltpu.make_async_copy(qc_hbm.at[0, :, pl.ds(b0, _BPW)], cidx_v.at[j],
                              sem_ci.at[j]).wait()
        pltpu.make_async_copy(wtab_hbm.at[pl.ds(0, _BPW), :],
                              wbuf_v.at[j], sem_g.at[j]).wait()

    def fire_out(q, j):
        pltpu.async_copy(cbuf_v.at[j], outc_hbm.at[q, :, pl.ds(b0, _BPW)],
                         sem_oc.at[j])
        pltpu.async_copy(wbuf_v.at[j, :, pl.ds(0, _DW)],
                         outw_hbm.at[q, pl.ds(b0, _BPW), :], sem_ow.at[j])

    def drain_out(j):
        pltpu.make_async_copy(cbuf_v.at[j], outc_hbm.at[0, :, pl.ds(b0, _BPW)],
                              sem_oc.at[j]).wait()
        pltpu.make_async_copy(wbuf_v.at[j, :, pl.ds(0, _DW)],
                              outw_hbm.at[0, pl.ds(b0, _BPW), :],
                              sem_ow.at[j]).wait()

    def compute_chunk(q, cidx, wbuf, cbuf):
        qv = jnp.full((_L,), q, jnp.int32)

        def group_body(g, carry):
            goff = g * _L
            lenv = len_v[pl.ds(goff, _L)]
            valid = qv < lenv
            maskf = jnp.where(valid, 1.0, 0.0).astype(jnp.float32)

            acc = [None] * _DC
            for w in range(_W):
                cw = cidx[w, pl.ds(goff, _L)]
                for d in range(_DC):
                    v = plsc.load_gather(ctab_v, [_splat(d), cw])
                    acc[d] = v if acc[d] is None else jnp.maximum(acc[d], v)
            for d in range(_DC):
                cbuf[d, pl.ds(goff, _L)] = acc[d] * maskf

            invalid = jnp.logical_not(valid)
            b_rel = goff + lane
            for k in range(_DW):
                col = jnp.bitwise_and(_splat(k) + lane, _DW - 1)
                plsc.store_scatter(wbuf, [b_rel, col], zeros, mask=invalid)
            return carry

        lax.fori_loop(0, _BPW // _L, group_body, 0)

    fire_inputs(0, 0)

    def iter_body(i, carry):
        for j in range(_R):
            q = i * _R + j
            nj = (j + 1) % _R
            if j == _R - 1:
                drain_out(nj)
                @pl.when(i < _NITER - 1)
                def _fire():
                    fire_inputs(q + 1, nj)
            else:
                @pl.when(i > 0)
                def _drain():
                    drain_out(nj)
                fire_inputs(q + 1, nj)
            drain_inputs(j)
            compute_chunk(q, cidx_v.at[j], wbuf_v.at[j], cbuf_v.at[j])
            fire_out(q, j)
        return carry

    lax.fori_loop(0, _NITER, iter_body, 0)

    for j in range(1, _R):
        drain_out(j)


_sc_call = pl.kernel(
    _sc_body,
    out_type=(
        jax.ShapeDtypeStruct((_Q, _DC, _B), jnp.float32),
        jax.ShapeDtypeStruct((_Q, _B, _DW), jnp.float32),
    ),
    mesh=plsc.VectorSubcoreMesh(core_axis_name="c", subcore_axis_name="s"),
    scratch_types=[
        pltpu.VMEM((_DC, _VC), jnp.float32),
        pltpu.VMEM((_BPW,), jnp.int32),
        pltpu.VMEM((_Q, _BPW), jnp.int32),
        pltpu.VMEM((_R, _W, _BPW), jnp.int32),
        pltpu.VMEM((_R, _BPW, _DWP), jnp.float32),
        pltpu.VMEM((_R, _DC, _BPW), jnp.float32),
        pltpu.SemaphoreType.DMA((_R,)),
        pltpu.SemaphoreType.DMA((_R,)),
        pltpu.SemaphoreType.DMA((_R,)),
        pltpu.SemaphoreType.DMA((_R,)),
    ],
    compiler_params=pltpu.CompilerParams(use_tc_tiling_on_sc=False,
                                         needs_layout_passes=False),
)


def kernel(question_words, question_chars, question_len, word_table, char_table):
    qw_t = question_words.astype(jnp.int32).T
    qc_t = question_chars.astype(jnp.int32).transpose(1, 2, 0)
    ctab_t = char_table.T
    ql = question_len.astype(jnp.int32)
    wt128 = jnp.pad(word_table, ((0, 0), (0, _DWP - _DW)))
    out_c, out_w = _sc_call(qw_t, qc_t, ql, wt128, ctab_t)
    return jnp.concatenate(
        [out_c.transpose(2, 0, 1), out_w.transpose(1, 0, 2)], axis=2)


# speedup vs baseline: 13.3829x; 1.0282x over previous
import jax
import jax.numpy as jnp
from jax import lax
from jax.experimental import pallas as pl
from jax.experimental.pallas import tpu as pltpu
from jax.experimental.pallas import tpu_sc as plsc

_B, _Q, _W = 4096, 50, 16
_DW, _DC = 64, 16
_DWP = 128
_VC = 512
_NW = 32
_BPW = _B // _NW
_CH = 64
_NT = (_DC + _DW) // 8
_R = 4
_NCH = _Q * (_BPW // _CH)
_NITER = _NCH // _R
_L = 16


def _splat(v, dtype=jnp.int32):
    return jnp.full((_L,), v, dtype)


def _sc_body(qw_hbm, qc_hbm, qlen_hbm, wtab_hbm, ctab_hbm, out_hbm,
             ctab_v, len_v, widx_v, cidx_v, wbuf_v, wp_v, obuf_v,
             sem_g, sem_ci, sem_o):
    nc = 2
    wid = lax.axis_index("s") * nc + lax.axis_index("c")
    b0 = wid * _BPW

    pltpu.sync_copy(ctab_hbm, ctab_v)
    pltpu.sync_copy(qlen_hbm.at[pl.ds(b0, _BPW)], len_v)
    pltpu.sync_copy(qw_hbm.at[:, pl.ds(b0, _BPW)], widx_v)

    lane = lax.iota(jnp.int32, _L)

    def fire_inputs(q, boff, j):
        pltpu.async_copy(qc_hbm.at[q, :, pl.ds(b0 + boff, _CH)], cidx_v.at[j],
                         sem_ci.at[j])
        pltpu.async_copy(wtab_hbm.at[widx_v.at[q, pl.ds(boff, _CH)]],
                         wbuf_v.at[j], sem_g.at[j])

    def drain_inputs(j):
        pltpu.make_async_copy(qc_hbm.at[0, :, pl.ds(b0, _CH)], cidx_v.at[j],
                              sem_ci.at[j]).wait()
        pltpu.make_async_copy(wtab_hbm.at[pl.ds(0, _CH), :],
                              wbuf_v.at[j], sem_g.at[j]).wait()

    def fire_out(q, boff, j):
        pltpu.async_copy(obuf_v.at[j],
                         out_hbm.at[q, :, wid, :, pl.ds(boff, _CH)],
                         sem_o.at[j])

    def drain_out(j):
        pltpu.make_async_copy(obuf_v.at[j],
                              out_hbm.at[0, :, wid, :, pl.ds(0, _CH)],
                              sem_o.at[j]).wait()

    def compute_chunk(q, boff, cidx, wbuf, obuf):
        qv = jnp.full((_L,), q, jnp.int32)

        def pos_body(p, carry):
            for k in range(_DW // _L):
                wp_v[p, pl.ds(k * _L, _L)] = wbuf[p, pl.ds(k * _L, _L)]
            return carry

        lax.fori_loop(0, _CH, pos_body, 0)

        def group_body(g, carry):
            goff = g * _L
            lenv = len_v[pl.ds(boff + goff, _L)]
            valid = qv < lenv
            maskf = jnp.where(valid, 1.0, 0.0).astype(jnp.float32)

            acc = [None] * _DC
            for w in range(_W):
                cw = cidx[w, pl.ds(goff, _L)]
                for d in range(_DC):
                    v = plsc.load_gather(ctab_v, [_splat(d), cw])
                    acc[d] = v if acc[d] is None else jnp.maximum(acc[d], v)
            for d in range(_DC):
                obuf[d // 8, d % 8, pl.ds(goff, _L)] = acc[d] * maskf

            b_rel = goff + lane
            for dd in range(_DW):
                v = plsc.load_gather(wp_v, [b_rel, _splat(dd)])
                dout = _DC + dd
                obuf[dout // 8, dout % 8, pl.ds(goff, _L)] = v * maskf
            return carry

        lax.fori_loop(0, _CH // _L, group_body, 0)

    fire_inputs(0, 0, 0)

    def iter_body(i, carry):
        for j in range(_R):
            c = i * _R + j
            q = i * 2 + (j // 2)
            boff = (j % 2) * _CH
            nq = q + (j % 2)
            nboff = ((j + 1) % 2) * _CH
            nj = (j + 1) % _R
            if j == _R - 1:
                drain_out(nj)
                @pl.when(i < _NITER - 1)
                def _fire():
                    fire_inputs(nq, nboff, nj)
            else:
                @pl.when(i > 0)
                def _drain():
                    drain_out(nj)
                fire_inputs(nq, nboff, nj)
            drain_inputs(j)
            compute_chunk(q, boff, cidx_v.at[j], wbuf_v.at[j], obuf_v.at[j])
            fire_out(q, boff, j)
        return carry

    lax.fori_loop(0, _NITER, iter_body, 0)

    for j in range(1, _R):
        drain_out(j)


_sc_call = pl.kernel(
    _sc_body,
    out_type=jax.ShapeDtypeStruct((_Q, _NT, _NW, 8, _BPW), jnp.float32),
    mesh=plsc.VectorSubcoreMesh(core_axis_name="c", subcore_axis_name="s"),
    scratch_types=[
        pltpu.VMEM((_DC, _VC), jnp.float32),
        pltpu.VMEM((_BPW,), jnp.int32),
        pltpu.VMEM((_Q, _BPW), jnp.int32),
        pltpu.VMEM((_R, _W, _CH), jnp.int32),
        pltpu.VMEM((_R, _CH, _DWP), jnp.float32),
        pltpu.VMEM((_CH, _DW + 1), jnp.float32),
        pltpu.VMEM((_R, _NT, 8, _CH), jnp.float32),
        pltpu.SemaphoreType.DMA((_R,)),
        pltpu.SemaphoreType.DMA((_R,)),
        pltpu.SemaphoreType.DMA((_R,)),
    ],
    compiler_params=pltpu.CompilerParams(use_tc_tiling_on_sc=False,
                                         needs_layout_passes=False),
)


def kernel(question_words, question_chars, question_len, word_table, char_table):
    qw_t = question_words.astype(jnp.int32).T
    qc_t = question_chars.astype(jnp.int32).transpose(1, 2, 0)
    ctab_t = char_table.T
    ql = question_len.astype(jnp.int32)
    wt128 = jnp.pad(word_table, ((0, 0), (0, _DWP - _DW)))
    out5 = _sc_call(qw_t, qc_t, ql, wt128, ctab_t)
    return out5.transpose(2, 4, 0, 1, 3).reshape(_B, _Q, _DC + _DW)
